# baseline (speedup 1.0000x reference)
import functools

import jax
import jax.numpy as jnp
from jax import lax
from jax.experimental import pallas as pl
from jax.experimental.pallas import tpu as pltpu
from jax.experimental.pallas import tpu_sc as plsc

N = 32768
D = 512
C = 10
S = 16
BN = 4096
GRID = N // BN

_SC_INFO = plsc.get_sparse_core_info()
NC = _SC_INFO.num_cores
NS = _SC_INFO.num_subcores
NW = NC * NS
NWATOMS = N // NW


def _tc_dense(emb_ref, bi_ref, w_ref, b_ref, hact_ref, sums_ref):
    i = pl.program_id(0)
    h = jax.lax.dot_general(
        w_ref[...], emb_ref[...], (((0,), (1,)), ((), ())),
        preferred_element_type=jnp.float32) + b_ref[...]
    row = jax.lax.broadcasted_iota(jnp.int32, (2 * C, BN), 0)
    hact = jnp.where(row < C, jax.nn.softplus(h), h)
    hact_ref[...] = hact

    bi = bi_ref[:, pl.ds(i * BN, BN)]
    oh = (bi == jax.lax.broadcasted_iota(jnp.int32, (S, BN), 0)
          ).astype(jnp.float32)
    part = jax.lax.dot_general(
        hact, oh, (((1,), (1,)), ((), ())),
        preferred_element_type=jnp.float32)

    @pl.when(i == 0)
    def _init():
        sums_ref[...] = part

    @pl.when(i != 0)
    def _acc():
        sums_ref[...] += part


@functools.partial(
    pl.kernel,
    mesh=plsc.VectorSubcoreMesh(core_axis_name="c", subcore_axis_name="s"),
    out_type=jax.ShapeDtypeStruct((C, N), jnp.float32),
    scratch_types=[
        pltpu.VMEM((NWATOMS,), jnp.int32),
        pltpu.VMEM((2 * C, S), jnp.float32),
        pltpu.VMEM((S,), jnp.float32),
        pltpu.VMEM((2 * C, NWATOMS), jnp.float32),
        pltpu.VMEM((C, NWATOMS), jnp.float32),
    ],
)
def _sc_combine(hact_hbm, sums_hbm, qtot_hbm, bi_hbm, q_hbm,
                bi_v, sums_v, qtot_v, row_v, q_v):
    wid = lax.axis_index("s") * NC + lax.axis_index("c")
    base = wid * NWATOMS
    pltpu.sync_copy(bi_hbm.at[pl.ds(base, NWATOMS)], bi_v)
    pltpu.sync_copy(sums_hbm, sums_v)
    pltpu.sync_copy(qtot_hbm, qtot_v)
    for r in range(2 * C):
        pltpu.sync_copy(hact_hbm.at[r, pl.ds(base, NWATOMS)], row_v.at[r])

    qtot = qtot_v[...]
    zero = jnp.zeros((S,), jnp.float32)
    one = jnp.ones((S,), jnp.float32)
    fsys = []
    for c in range(C):
        ws = sums_v[c, :]
        qs = sums_v[C + c, :]
        good = ws > zero
        fsys.append(jnp.where(
            good, (qtot - qs) / jnp.where(good, ws, one), zero))

    gdn = lax.GatherDimensionNumbers(
        offset_dims=(), collapsed_slice_dims=(0,), start_index_map=(0,))

    def body(j, carry):
        sl = pl.ds(j * 16, 16)
        sid = bi_v[sl]
        for c in range(C):
            f = lax.gather(fsys[c], sid[:, None], gdn, (1,),
                           mode=lax.GatherScatterMode.PROMISE_IN_BOUNDS)
            q_v[c, sl] = row_v[C + c, sl] + row_v[c, sl] * f
        return carry

    lax.fori_loop(0, NWATOMS // 16, body, 0)

    for c in range(C):
        pltpu.sync_copy(q_v.at[c], q_hbm.at[c, pl.ds(base, NWATOMS)])


@jax.jit
def _run(embedding, batch_index, total_charge, W_wi, b_wi, W_qi, b_qi):
    bi_row = batch_index.reshape(1, N)
    w_cat = jnp.concatenate([W_wi, W_qi], axis=1)
    b_cat = jnp.concatenate([b_wi, b_qi]).reshape(2 * C, 1)

    hact, sums = pl.pallas_call(
        _tc_dense,
        grid=(GRID,),
        in_specs=[
            pl.BlockSpec((BN, D), lambda i: (i, 0)),
            pl.BlockSpec((1, N), lambda i: (0, 0)),
            pl.BlockSpec((D, 2 * C), lambda i: (0, 0)),
            pl.BlockSpec((2 * C, 1), lambda i: (0, 0)),
        ],
        out_specs=[
            pl.BlockSpec((2 * C, BN), lambda i: (0, i)),
            pl.BlockSpec((2 * C, S), lambda i: (0, 0)),
        ],
        out_shape=[
            jax.ShapeDtypeStruct((2 * C, N), jnp.float32),
            jax.ShapeDtypeStruct((2 * C, S), jnp.float32),
        ],
    )(embedding, bi_row, w_cat, b_cat)

    q_t = _sc_combine(hact, sums, total_charge, batch_index)
    return q_t.T


def kernel(embedding, coordinates, batch_index, natoms, total_charge,
           W_wi, b_wi, W_qi, b_qi):
    del coordinates, natoms
    return _run(embedding.astype(jnp.float32), batch_index,
                total_charge.astype(jnp.float32), W_wi, b_wi, W_qi, b_qi)


# speedup vs baseline: 1.2020x; 1.2020x over previous
import functools

import jax
import jax.numpy as jnp
from jax import lax
from jax.experimental import pallas as pl
from jax.experimental.pallas import tpu as pltpu
from jax.experimental.pallas import tpu_sc as plsc

N = 32768
D = 512
C = 10
S = 16
BN = 4096
GRID = N // BN

_SC_INFO = plsc.get_sparse_core_info()
NC = _SC_INFO.num_cores
NS = _SC_INFO.num_subcores
NW = NC * NS
NWATOMS = N // NW


def _tc_dense(emb_ref, bi_ref, w_ref, b_ref, hact_ref, sums_ref):
    i = pl.program_id(0)
    h = jax.lax.dot_general(
        w_ref[...], emb_ref[...], (((0,), (1,)), ((), ())),
        preferred_element_type=jnp.float32) + b_ref[...]
    row = jax.lax.broadcasted_iota(jnp.int32, (2 * C, BN), 0)
    hact = jnp.where(row < C, jax.nn.softplus(h), h)
    hact_ref[...] = hact

    bi = bi_ref[:, pl.ds(i * BN, BN)]
    oh = (bi == jax.lax.broadcasted_iota(jnp.int32, (S, BN), 0)
          ).astype(jnp.float32)
    part = jax.lax.dot_general(
        hact, oh, (((1,), (1,)), ((), ())),
        preferred_element_type=jnp.float32)

    @pl.when(i == 0)
    def _init():
        sums_ref[...] = part

    @pl.when(i != 0)
    def _acc():
        sums_ref[...] += part


@functools.partial(
    pl.kernel,
    mesh=plsc.VectorSubcoreMesh(core_axis_name="c", subcore_axis_name="s"),
    out_type=jax.ShapeDtypeStruct((C, N), jnp.float32),
    scratch_types=[
        pltpu.VMEM((NWATOMS,), jnp.int32),
        pltpu.VMEM((2 * C, S), jnp.float32),
        pltpu.VMEM((S,), jnp.float32),
        pltpu.VMEM((2 * C, NWATOMS), jnp.float32),
        pltpu.VMEM((C, NWATOMS), jnp.float32),
    ],
)
def _sc_combine(hact_hbm, sums_hbm, qtot_hbm, bi_hbm, q_hbm,
                bi_v, sums_v, qtot_v, row_v, q_v):
    wid = lax.axis_index("s") * NC + lax.axis_index("c")
    base = wid * NWATOMS
    pltpu.sync_copy(bi_hbm.at[pl.ds(base, NWATOMS)], bi_v)
    pltpu.sync_copy(sums_hbm, sums_v)
    pltpu.sync_copy(qtot_hbm, qtot_v)
    pltpu.sync_copy(hact_hbm.at[:, pl.ds(base, NWATOMS)], row_v)

    qtot = qtot_v[...]
    zero = jnp.zeros((S,), jnp.float32)
    one = jnp.ones((S,), jnp.float32)
    fsys = []
    for c in range(C):
        ws = sums_v[c, :]
        qs = sums_v[C + c, :]
        good = ws > zero
        fsys.append(jnp.where(
            good, (qtot - qs) / jnp.where(good, ws, one), zero))

    gdn = lax.GatherDimensionNumbers(
        offset_dims=(), collapsed_slice_dims=(0,), start_index_map=(0,))

    def body(j, carry):
        sl = pl.ds(j * 16, 16)
        sid = bi_v[sl]
        for c in range(C):
            f = lax.gather(fsys[c], sid[:, None], gdn, (1,),
                           mode=lax.GatherScatterMode.PROMISE_IN_BOUNDS)
            q_v[c, sl] = row_v[C + c, sl] + row_v[c, sl] * f
        return carry

    lax.fori_loop(0, NWATOMS // 16, body, 0)

    pltpu.sync_copy(q_v, q_hbm.at[:, pl.ds(base, NWATOMS)])


@jax.jit
def _run(embedding, batch_index, total_charge, W_wi, b_wi, W_qi, b_qi):
    bi_row = batch_index.reshape(1, N)
    w_cat = jnp.concatenate([W_wi, W_qi], axis=1)
    b_cat = jnp.concatenate([b_wi, b_qi]).reshape(2 * C, 1)

    hact, sums = pl.pallas_call(
        _tc_dense,
        grid=(GRID,),
        in_specs=[
            pl.BlockSpec((BN, D), lambda i: (i, 0)),
            pl.BlockSpec((1, N), lambda i: (0, 0)),
            pl.BlockSpec((D, 2 * C), lambda i: (0, 0)),
            pl.BlockSpec((2 * C, 1), lambda i: (0, 0)),
        ],
        out_specs=[
            pl.BlockSpec((2 * C, BN), lambda i: (0, i)),
            pl.BlockSpec((2 * C, S), lambda i: (0, 0)),
        ],
        out_shape=[
            jax.ShapeDtypeStruct((2 * C, N), jnp.float32),
            jax.ShapeDtypeStruct((2 * C, S), jnp.float32),
        ],
    )(embedding, bi_row, w_cat, b_cat)

    q_t = _sc_combine(hact, sums, total_charge, batch_index)
    return q_t.T


def kernel(embedding, coordinates, batch_index, natoms, total_charge,
           W_wi, b_wi, W_qi, b_qi):
    del coordinates, natoms
    return _run(embedding.astype(jnp.float32), batch_index,
                total_charge.astype(jnp.float32), W_wi, b_wi, W_qi, b_qi)
